# baseline (speedup 1.0000x reference)
import jax
import jax.numpy as jnp
from jax import lax
from jax.experimental import pallas as pl

_DN = (((0,), (0,)), ((), ()))


def _body(x_ref, eT_ref, u_ref, Wn_ref, bn_ref, We_ref, be_ref, Wg_ref, bg_ref,
          xo_ref, eo_ref, uo_ref):
    i = pl.program_id(0)
    eo_ref[...] = (
        lax.dot_general(eT_ref[...].astype(jnp.bfloat16),
                        We_ref[...].astype(jnp.bfloat16), _DN,
                        preferred_element_type=jnp.float32)
        + be_ref[...]
    )
    xo_ref[...] = (
        jnp.dot(x_ref[...], Wn_ref[...], preferred_element_type=jnp.float32)
        + bn_ref[...]
    )

    @pl.when(i == 0)
    def _():
        uo_ref[...] = (
            jnp.dot(u_ref[...], Wg_ref[...], preferred_element_type=jnp.float32)
            + bg_ref[...]
        )


def kernel(x, edge_attr, u, W_node, b_node, W_edge, b_edge, W_glob, b_glob):
    n_nodes, d_node = x.shape
    n_edges, d_edge = edge_attr.shape
    latent = W_node.shape[1]

    eT = edge_attr.T

    eblk = min(n_edges, 48128)
    grid = pl.cdiv(n_edges, eblk)
    nblk = max(8, 8 * pl.cdiv(pl.cdiv(n_nodes, grid), 8))

    bn = b_node.reshape(1, latent)
    be = b_edge.reshape(1, latent)
    bg = b_glob.reshape(1, latent)

    x_emb, edge_emb, u_emb = pl.pallas_call(
        _body,
        grid=(grid,),
        in_specs=[
            pl.BlockSpec((nblk, d_node), lambda i: (i, 0)),
            pl.BlockSpec((d_edge, eblk), lambda i: (0, i)),
            pl.BlockSpec((1, u.shape[1]), lambda i: (0, 0)),
            pl.BlockSpec((d_node, latent), lambda i: (0, 0)),
            pl.BlockSpec((1, latent), lambda i: (0, 0)),
            pl.BlockSpec((d_edge, latent), lambda i: (0, 0)),
            pl.BlockSpec((1, latent), lambda i: (0, 0)),
            pl.BlockSpec((u.shape[1], latent), lambda i: (0, 0)),
            pl.BlockSpec((1, latent), lambda i: (0, 0)),
        ],
        out_specs=[
            pl.BlockSpec((nblk, latent), lambda i: (i, 0)),
            pl.BlockSpec((eblk, latent), lambda i: (i, 0)),
            pl.BlockSpec((1, latent), lambda i: (0, 0)),
        ],
        out_shape=[
            jax.ShapeDtypeStruct((n_nodes, latent), jnp.float32),
            jax.ShapeDtypeStruct((n_edges, latent), jnp.float32),
            jax.ShapeDtypeStruct((1, latent), jnp.float32),
        ],
    )(x, eT, u, W_node, bn, W_edge, be, W_glob, bg)
    return (x_emb, edge_emb, u_emb)


# speedup vs baseline: 1.0011x; 1.0011x over previous
import jax
import jax.numpy as jnp
from jax import lax
from jax.experimental import pallas as pl

_DN = (((0,), (0,)), ((), ()))


def _body(x_ref, eT_ref, u_ref, Wn_ref, bn_ref, We_ref, be_ref, Wg_ref, bg_ref,
          xo_ref, eo_ref, uo_ref):
    i = pl.program_id(0)
    eo_ref[...] = (
        lax.dot_general(eT_ref[...].astype(jnp.bfloat16),
                        We_ref[...].astype(jnp.bfloat16), _DN,
                        preferred_element_type=jnp.float32)
        + be_ref[...]
    )
    xo_ref[...] = (
        jnp.dot(x_ref[...], Wn_ref[...], preferred_element_type=jnp.float32)
        + bn_ref[...]
    )

    @pl.when(i == 0)
    def _():
        uo_ref[...] = (
            jnp.dot(u_ref[...], Wg_ref[...], preferred_element_type=jnp.float32)
            + bg_ref[...]
        )


def kernel(x, edge_attr, u, W_node, b_node, W_edge, b_edge, W_glob, b_glob):
    n_nodes, d_node = x.shape
    n_edges, d_edge = edge_attr.shape
    latent = W_node.shape[1]

    eT = edge_attr.T

    eblk = min(n_edges, 47104)
    grid = pl.cdiv(n_edges, eblk)
    nblk = max(8, 8 * pl.cdiv(pl.cdiv(n_nodes, grid), 8))

    bn = b_node.reshape(1, latent)
    be = b_edge.reshape(1, latent)
    bg = b_glob.reshape(1, latent)

    x_emb, edge_emb, u_emb = pl.pallas_call(
        _body,
        grid=(grid,),
        in_specs=[
            pl.BlockSpec((nblk, d_node), lambda i: (i, 0)),
            pl.BlockSpec((d_edge, eblk), lambda i: (0, i)),
            pl.BlockSpec((1, u.shape[1]), lambda i: (0, 0)),
            pl.BlockSpec((d_node, latent), lambda i: (0, 0)),
            pl.BlockSpec((1, latent), lambda i: (0, 0)),
            pl.BlockSpec((d_edge, latent), lambda i: (0, 0)),
            pl.BlockSpec((1, latent), lambda i: (0, 0)),
            pl.BlockSpec((u.shape[1], latent), lambda i: (0, 0)),
            pl.BlockSpec((1, latent), lambda i: (0, 0)),
        ],
        out_specs=[
            pl.BlockSpec((nblk, latent), lambda i: (i, 0)),
            pl.BlockSpec((eblk, latent), lambda i: (i, 0)),
            pl.BlockSpec((1, latent), lambda i: (0, 0)),
        ],
        out_shape=[
            jax.ShapeDtypeStruct((n_nodes, latent), jnp.float32),
            jax.ShapeDtypeStruct((n_edges, latent), jnp.float32),
            jax.ShapeDtypeStruct((1, latent), jnp.float32),
        ],
    )(x, eT, u, W_node, bn, W_edge, be, W_glob, bg)
    return (x_emb, edge_emb, u_emb)
